# baseline (speedup 1.0000x reference)
import functools

import jax
import jax.numpy as jnp
from jax import lax
from jax.experimental import pallas as pl
from jax.experimental.pallas import tpu as pltpu
from jax.experimental.pallas import tpu_sc as plsc

_N, _K, _D = 10000, 32, 128
_NC, _NS = 2, 16
_NW = _NC * _NS
_G = 8
_NBLK = _N // _G
_LANES = 16
_DV = _D // _LANES


def _sc_reduce(nf):
    mesh = plsc.VectorSubcoreMesh(
        core_axis_name="c", subcore_axis_name="s", num_cores=_NC, num_subcores=_NS
    )

    @functools.partial(
        pl.kernel,
        out_type=jax.ShapeDtypeStruct((_N, _D), jnp.float32),
        mesh=mesh,
        scratch_types=[
            pltpu.VMEM((2, _G, _K, _D), jnp.float32),
            pltpu.VMEM((2, _G, _D), jnp.float32),
            pltpu.SemaphoreType.DMA((2,)),
            pltpu.SemaphoreType.DMA((2,)),
        ],
    )
    def sc_kernel(nf_hbm, out_hbm, buf, ostage, insem, osem):
        wid = lax.axis_index("s") * _NC + lax.axis_index("c")
        nmy = (_NBLK - wid + _NW - 1) // _NW

        def in_copy(i, slot):
            b = wid + i * _NW
            return pltpu.make_async_copy(
                nf_hbm.at[pl.ds(b * _G, _G)], buf.at[slot], insem.at[slot]
            )

        def out_copy(i, slot):
            b = wid + i * _NW
            return pltpu.make_async_copy(
                ostage.at[slot], out_hbm.at[pl.ds(b * _G, _G)], osem.at[slot]
            )

        @pl.when(nmy > 0)
        def _():
            in_copy(0, 0).start()

        def block_body(i, carry):
            slot = lax.rem(i, 2)
            in_copy(i, slot).wait()

            @pl.when(i + 1 < nmy)
            def _():
                in_copy(i + 1, 1 - slot).start()

            @pl.when(i >= 2)
            def _():
                out_copy(i - 2, slot).wait()

            def node_body(nn, c):
                accs = [buf[slot, nn, 0, pl.ds(d * _LANES, _LANES)] for d in range(_DV)]
                for k in range(1, _K):
                    for d in range(_DV):
                        accs[d] = accs[d] + buf[slot, nn, k, pl.ds(d * _LANES, _LANES)]
                for d in range(_DV):
                    ostage[slot, nn, pl.ds(d * _LANES, _LANES)] = accs[d] * (1.0 / _K)
                return c

            lax.fori_loop(0, _G, node_body, 0)
            out_copy(i, slot).start()
            return carry

        lax.fori_loop(0, nmy, block_body, 0)

        @pl.when(nmy >= 2)
        def _():
            out_copy(nmy - 2, lax.rem(nmy, 2)).wait()

        @pl.when(nmy >= 1)
        def _():
            out_copy(nmy - 1, lax.rem(nmy - 1, 2)).wait()

    return sc_kernel(nf)


_MR = 1000


def _mm_body(agg_ref, w_ref, out_ref):
    out_ref[...] = jnp.dot(agg_ref[...], w_ref[...], preferred_element_type=jnp.float32)


def _tc_matmul(agg, weight):
    return pl.pallas_call(
        _mm_body,
        grid=(_N // _MR,),
        in_specs=[
            pl.BlockSpec((_MR, _D), lambda i: (i, 0)),
            pl.BlockSpec((_D, _D), lambda i: (0, 0)),
        ],
        out_specs=pl.BlockSpec((_MR, _D), lambda i: (i, 0)),
        out_shape=jax.ShapeDtypeStruct((_N, _D), jnp.float32),
    )(agg, weight)


def kernel(neighbor_feature, weight):
    agg = _sc_reduce(neighbor_feature)
    return _tc_matmul(agg, weight)


# speedup vs baseline: 1.4490x; 1.4490x over previous
import functools

import jax
import jax.numpy as jnp
from jax import lax
from jax.experimental import pallas as pl
from jax.experimental.pallas import tpu as pltpu
from jax.experimental.pallas import tpu_sc as plsc

_N, _K, _D = 10000, 32, 128
_NC, _NS = 2, 16
_NW = _NC * _NS
_G = 8
_LANES = 16
_DV = _D // _LANES

_R = 200
_S = 7200


def _sc_reduce(nf, base, rows):
    nblk = rows // _G
    mesh = plsc.VectorSubcoreMesh(
        core_axis_name="c", subcore_axis_name="s", num_cores=_NC, num_subcores=_NS
    )

    @functools.partial(
        pl.kernel,
        out_type=jax.ShapeDtypeStruct((rows, _D), jnp.float32),
        mesh=mesh,
        scratch_types=[
            pltpu.VMEM((2, _G, _K, _D), jnp.float32),
            pltpu.VMEM((2, _G, _D), jnp.float32),
            pltpu.SemaphoreType.DMA((2,)),
            pltpu.SemaphoreType.DMA((2,)),
        ],
    )
    def sc_kernel(nf_hbm, out_hbm, buf, ostage, insem, osem):
        wid = lax.axis_index("s") * _NC + lax.axis_index("c")
        nmy = (nblk - wid + _NW - 1) // _NW

        def in_copy(i, slot):
            b = wid + i * _NW
            return pltpu.make_async_copy(
                nf_hbm.at[pl.ds(base + b * _G, _G)], buf.at[slot], insem.at[slot]
            )

        def out_copy(i, slot):
            b = wid + i * _NW
            return pltpu.make_async_copy(
                ostage.at[slot], out_hbm.at[pl.ds(b * _G, _G)], osem.at[slot]
            )

        @pl.when(nmy > 0)
        def _():
            in_copy(0, 0).start()

        def block_body(i, carry):
            slot = lax.rem(i, 2)
            in_copy(i, slot).wait()

            @pl.when(i + 1 < nmy)
            def _():
                in_copy(i + 1, 1 - slot).start()

            @pl.when(i >= 2)
            def _():
                out_copy(i - 2, slot).wait()

            def node_body(nn, c):
                accs = [buf[slot, nn, 0, pl.ds(d * _LANES, _LANES)] for d in range(_DV)]
                for k in range(1, _K):
                    for d in range(_DV):
                        accs[d] = accs[d] + buf[slot, nn, k, pl.ds(d * _LANES, _LANES)]
                for d in range(_DV):
                    ostage[slot, nn, pl.ds(d * _LANES, _LANES)] = accs[d] * (1.0 / _K)
                return c

            lax.fori_loop(0, _G, node_body, 0)
            out_copy(i, slot).start()
            return carry

        lax.fori_loop(0, nmy, block_body, 0)

        @pl.when(nmy >= 2)
        def _():
            out_copy(nmy - 2, lax.rem(nmy, 2)).wait()

        @pl.when(nmy >= 1)
        def _():
            out_copy(nmy - 1, lax.rem(nmy - 1, 2)).wait()

    return sc_kernel(nf)


def _fused_body(nf_ref, w_ref, out_ref):
    agg = jnp.sum(nf_ref[...], axis=1) * (1.0 / _K)
    out_ref[...] = jnp.dot(agg, w_ref[...], preferred_element_type=jnp.float32)


def _tc_fused(nf, weight, rows):
    return pl.pallas_call(
        _fused_body,
        grid=(rows // _R,),
        in_specs=[
            pl.BlockSpec((_R, _K, _D), lambda i: (i, 0, 0)),
            pl.BlockSpec((_D, _D), lambda i: (0, 0)),
        ],
        out_specs=pl.BlockSpec((_R, _D), lambda i: (i, 0)),
        out_shape=jax.ShapeDtypeStruct((rows, _D), jnp.float32),
    )(nf, weight)


def _mm_body(agg_ref, w_ref, out_ref):
    out_ref[...] = jnp.dot(agg_ref[...], w_ref[...], preferred_element_type=jnp.float32)


def _tc_matmul(agg, weight, rows):
    return pl.pallas_call(
        _mm_body,
        grid=(rows // _R,),
        in_specs=[
            pl.BlockSpec((_R, _D), lambda i: (i, 0)),
            pl.BlockSpec((_D, _D), lambda i: (0, 0)),
        ],
        out_specs=pl.BlockSpec((_R, _D), lambda i: (i, 0)),
        out_shape=jax.ShapeDtypeStruct((rows, _D), jnp.float32),
    )(agg, weight)


def kernel(neighbor_feature, weight):
    agg_b = _sc_reduce(neighbor_feature, _S, _N - _S)
    out_a = _tc_fused(neighbor_feature, weight, _S)
    out_b = _tc_matmul(agg_b, weight, _N - _S)
    return jnp.concatenate([out_a, out_b], axis=0)


# speedup vs baseline: 1.9525x; 1.3475x over previous
import jax
import jax.numpy as jnp
from jax.experimental import pallas as pl

_N, _K, _D = 10000, 32, 128
_R = 200


def _fused_body(nf_ref, w_ref, out_ref):
    agg = jnp.sum(nf_ref[...], axis=1) * (1.0 / _K)
    out_ref[...] = jnp.dot(agg, w_ref[...], preferred_element_type=jnp.float32)


def kernel(neighbor_feature, weight):
    return pl.pallas_call(
        _fused_body,
        grid=(_N // _R,),
        in_specs=[
            pl.BlockSpec((_R, _K, _D), lambda i: (i, 0, 0)),
            pl.BlockSpec((_D, _D), lambda i: (0, 0)),
        ],
        out_specs=pl.BlockSpec((_R, _D), lambda i: (i, 0)),
        out_shape=jax.ShapeDtypeStruct((_N, _D), jnp.float32),
    )(neighbor_feature, weight)


# speedup vs baseline: 2.3523x; 1.2048x over previous
import jax
import jax.numpy as jnp
from jax.experimental import pallas as pl

_N, _K, _D = 10000, 32, 128
_R = 1000


def _fused_body(nf_ref, w_ref, out_ref):
    agg = jnp.sum(nf_ref[...], axis=1) * (1.0 / _K)
    out_ref[...] = jnp.dot(agg, w_ref[...], preferred_element_type=jnp.float32)


def kernel(neighbor_feature, weight):
    return pl.pallas_call(
        _fused_body,
        grid=(_N // _R,),
        in_specs=[
            pl.BlockSpec((_R, _K, _D), lambda i: (i, 0, 0)),
            pl.BlockSpec((_D, _D), lambda i: (0, 0)),
        ],
        out_specs=pl.BlockSpec((_R, _D), lambda i: (i, 0)),
        out_shape=jax.ShapeDtypeStruct((_N, _D), jnp.float32),
    )(neighbor_feature, weight)


# speedup vs baseline: 2.4360x; 1.0356x over previous
import jax
import jax.numpy as jnp
from jax.experimental import pallas as pl

_N, _K, _D = 10000, 32, 128
_R = 400


def _fused_body(nf_ref, w_ref, out_ref):
    agg = jnp.sum(nf_ref[...], axis=1) * (1.0 / _K)
    out_ref[...] = jnp.dot(agg, w_ref[...], preferred_element_type=jnp.float32)


def kernel(neighbor_feature, weight):
    return pl.pallas_call(
        _fused_body,
        grid=(_N // _R,),
        in_specs=[
            pl.BlockSpec((_R, _K, _D), lambda i: (i, 0, 0)),
            pl.BlockSpec((_D, _D), lambda i: (0, 0)),
        ],
        out_specs=pl.BlockSpec((_R, _D), lambda i: (i, 0)),
        out_shape=jax.ShapeDtypeStruct((_N, _D), jnp.float32),
    )(neighbor_feature, weight)


# speedup vs baseline: 2.4410x; 1.0020x over previous
import jax
import jax.numpy as jnp
from jax.experimental import pallas as pl

_N, _K, _D = 10000, 32, 128
_R = 480


def _fused_body(nf_ref, w_ref, out_ref):
    agg = jnp.sum(nf_ref[...], axis=1) * (1.0 / _K)
    out_ref[...] = jnp.dot(agg, w_ref[...], preferred_element_type=jnp.float32)


def kernel(neighbor_feature, weight):
    return pl.pallas_call(
        _fused_body,
        grid=(pl.cdiv(_N, _R),),
        in_specs=[
            pl.BlockSpec((_R, _K, _D), lambda i: (i, 0, 0)),
            pl.BlockSpec((_D, _D), lambda i: (0, 0)),
        ],
        out_specs=pl.BlockSpec((_R, _D), lambda i: (i, 0)),
        out_shape=jax.ShapeDtypeStruct((_N, _D), jnp.float32),
    )(neighbor_feature, weight)
